# baseline (speedup 1.0000x reference)
import jax
import jax.numpy as jnp
import numpy as np
from jax.experimental import pallas as pl
from jax.experimental.pallas import tpu as pltpu

_ATOMS = 51
_V_MIN = -1.0
_V_MAX = 1.0


def _projection_matrix() -> np.ndarray:
    atoms = _ATOMS
    delta = (_V_MAX - _V_MIN) / (atoms - 1)
    supports = np.linspace(_V_MIN, _V_MAX, atoms).astype(np.float32)
    tz = np.clip(supports, _V_MIN, _V_MAX).astype(np.float32)
    b = ((tz - np.float32(_V_MIN)) / np.float32(delta)).astype(np.float32)
    l = np.floor(b).astype(np.int32)
    u = np.ceil(b).astype(np.int32)
    l = np.where((u > 0) & (l == u), l - 1, l)
    u = np.where((l < atoms - 1) & (l == u), u + 1, u)
    w_l = u.astype(np.float32) - b
    w_u = b - l.astype(np.float32)
    p = np.zeros((atoms, atoms), np.float32)
    np.add.at(p, (np.arange(atoms), l), w_l)
    np.add.at(p, (np.arange(atoms), u), w_u)
    return p


_P_CONST = _projection_matrix()


_CHUNK = 2048
_N_CHUNKS = 16384 // _CHUNK
_NBUF = 4


def _pipe_kernel(p_ref, a_hbm, f_hbm, out_ref, a_buf, f_buf, sem_a, sem_f):
    def _copy(i, slot, which_hbm, which_buf, sem):
        return pltpu.make_async_copy(
            which_hbm.at[:, pl.ds(i * _CHUNK, _CHUNK)],
            which_buf.at[slot],
            sem.at[slot],
        )

    def _start(i, slot):
        _copy(i, slot, a_hbm, a_buf, sem_a).start()
        _copy(i, slot, f_hbm, f_buf, sem_f).start()

    def _wait(i, slot):
        _copy(i, slot, a_hbm, a_buf, sem_a).wait()
        _copy(i, slot, f_hbm, f_buf, sem_f).wait()

    for j in range(_NBUF - 1):
        _start(j, j)
    acc = jnp.zeros((), jnp.float32)
    for i in range(_N_CHUNKS):
        slot = i % _NBUF
        if i + _NBUF - 1 < _N_CHUNKS:
            _start(i + _NBUF - 1, (i + _NBUF - 1) % _NBUF)
        _wait(i, slot)
        logf = jnp.log(f_buf[slot] + 1e-16)
        skewed = jax.lax.dot_general(
            p_ref[...], a_buf[slot],
            dimension_numbers=(((1,), (0,)), ((), ())),
            preferred_element_type=jnp.float32,
            precision=jax.lax.Precision.DEFAULT,
        )
        acc = acc + jnp.sum(skewed * logf)
    out_ref[0, 0] = acc


def kernel(anchor, feature):
    batch = anchor.shape[0]
    a_t = pltpu.with_memory_space_constraint(anchor.T, pltpu.MemorySpace.HBM)
    f_t = pltpu.with_memory_space_constraint(feature.T, pltpu.MemorySpace.HBM)
    p_t = jnp.asarray(_P_CONST.T.copy())
    acc = pl.pallas_call(
        _pipe_kernel,
        in_specs=[
            pl.BlockSpec(memory_space=pltpu.VMEM),
            pl.BlockSpec(memory_space=pltpu.MemorySpace.HBM),
            pl.BlockSpec(memory_space=pltpu.MemorySpace.HBM),
        ],
        out_specs=pl.BlockSpec(memory_space=pltpu.SMEM),
        out_shape=jax.ShapeDtypeStruct((1, 1), jnp.float32),
        scratch_shapes=[
            pltpu.VMEM((_NBUF, _ATOMS, _CHUNK), jnp.float32),
            pltpu.VMEM((_NBUF, _ATOMS, _CHUNK), jnp.float32),
            pltpu.SemaphoreType.DMA((_NBUF,)),
            pltpu.SemaphoreType.DMA((_NBUF,)),
        ],
    )(p_t, a_t, f_t)
    return -(acc[0, 0] / jnp.float32(batch))


# speedup vs baseline: 1.1190x; 1.1190x over previous
import jax
import jax.numpy as jnp
import numpy as np
from jax.experimental import pallas as pl
from jax.experimental.pallas import tpu as pltpu

_ATOMS = 51
_V_MIN = -1.0
_V_MAX = 1.0


def _projection_matrix() -> np.ndarray:
    atoms = _ATOMS
    delta = (_V_MAX - _V_MIN) / (atoms - 1)
    supports = np.linspace(_V_MIN, _V_MAX, atoms).astype(np.float32)
    tz = np.clip(supports, _V_MIN, _V_MAX).astype(np.float32)
    b = ((tz - np.float32(_V_MIN)) / np.float32(delta)).astype(np.float32)
    l = np.floor(b).astype(np.int32)
    u = np.ceil(b).astype(np.int32)
    l = np.where((u > 0) & (l == u), l - 1, l)
    u = np.where((l < atoms - 1) & (l == u), u + 1, u)
    w_l = u.astype(np.float32) - b
    w_u = b - l.astype(np.float32)
    p = np.zeros((atoms, atoms), np.float32)
    np.add.at(p, (np.arange(atoms), l), w_l)
    np.add.at(p, (np.arange(atoms), u), w_u)
    return p


_P_CONST = _projection_matrix()


def _loss_kernel(p_ref, a_ref, f_ref, out_ref):
    logf = jnp.log(f_ref[...] + 1e-16)
    skewed = jax.lax.dot_general(
        p_ref[...], a_ref[...],
        dimension_numbers=(((1,), (0,)), ((), ())),
        preferred_element_type=jnp.float32,
        precision=jax.lax.Precision.DEFAULT,
    )
    out_ref[0, 0] = jnp.sum(skewed * logf)


def kernel(anchor, feature):
    batch = anchor.shape[0]
    a_t = anchor.T
    f_t = feature.T
    p_t = jnp.asarray(_P_CONST.T.copy())
    acc = pl.pallas_call(
        _loss_kernel,
        in_specs=[
            pl.BlockSpec(memory_space=pltpu.VMEM),
            pl.BlockSpec(memory_space=pltpu.VMEM),
            pl.BlockSpec(memory_space=pltpu.VMEM),
        ],
        out_specs=pl.BlockSpec(memory_space=pltpu.SMEM),
        out_shape=jax.ShapeDtypeStruct((1, 1), jnp.float32),
    )(p_t, a_t, f_t)
    return -(acc[0, 0] / jnp.float32(batch))
